# Initial kernel scaffold; baseline (speedup 1.0000x reference)
import jax
import jax.numpy as jnp
from jax.experimental import pallas as pl


def kernel(x, edge_index, W1, b1, W2, b2, W3, b3, W4, b4, W5, b5):
    raise NotImplementedError("write your pallas kernel here")


# speedup vs baseline: 22.9169x; 22.9169x over previous
import functools

import jax
import jax.numpy as jnp
from jax import lax
from jax.experimental import pallas as pl
from jax.experimental.pallas import tpu as pltpu
from jax.experimental.pallas import tpu_sc as plsc

N = 10000
NP = 10240
E = 320000
NCLS = 13

_NC = 2
_NS = 16
_NW = _NC * _NS
_EPW = E // _NW
_CHUNK = 125
_T = _EPW // _CHUNK
_RPT = NP // _NS


def _fill_const(ref, val, nrow, f):
    cvec = jnp.full((16,), val, jnp.float32)

    def body(i, carry):
        for j in range(f // 16):
            ref[i, pl.ds(j * 16, 16)] = cvec
        return carry

    lax.fori_loop(0, nrow, body, 0)


def _sc_pass(u, srcr, dstr):
    f = u.shape[1]
    mesh = plsc.VectorSubcoreMesh(core_axis_name="c", subcore_axis_name="s")

    @functools.partial(
        pl.kernel,
        out_type=jax.ShapeDtypeStruct((_NC, NP, f), jnp.float32),
        mesh=mesh,
        scratch_types=[
            pltpu.VMEM((_T, _CHUNK), jnp.int32),
            pltpu.VMEM((_T, _CHUNK), jnp.int32),
            pltpu.VMEM((2, _CHUNK, f), jnp.float32),
            pltpu.VMEM((_RPT, f), jnp.float32),
            pltpu.VMEM_SHARED((NP, f), jnp.float32),
            pltpu.SemaphoreType.DMA,
            pltpu.SemaphoreType.DMA,
        ],
        compiler_params=pltpu.CompilerParams(use_tc_tiling_on_sc=False),
    )
    def k(u_hbm, src_hbm, dst_hbm, out_hbm, idx_s, idx_d, rows, rb, acc,
          g0, g1):
        c = lax.axis_index("c")
        s = lax.axis_index("s")
        wid = c * _NS + s
        gsems = (g0, g1)
        own = pl.ds(s * _RPT, _RPT)

        _fill_const(rb, 0.0, _RPT, f)
        pltpu.sync_copy(rb, acc.at[own])

        pltpu.sync_copy(src_hbm.at[wid], idx_s)
        pltpu.sync_copy(dst_hbm.at[wid], idx_d)

        plsc.subcore_barrier()

        pltpu.async_copy(u_hbm.at[idx_s.at[0]], rows.at[0], g0)
        pltpu.async_copy(u_hbm.at[idx_s.at[1]], rows.at[1], g1)

        def step(i, b, issue_next):
            pltpu.make_async_copy(u_hbm.at[idx_s.at[i]], rows.at[b],
                                  gsems[b]).wait()
            pltpu.sync_copy(rows.at[b], acc.at[idx_d.at[i]], add=True)
            if issue_next:
                pltpu.async_copy(u_hbm.at[idx_s.at[i + 2]], rows.at[b],
                                 gsems[b])

        def body(kk, carry):
            t = 2 * kk
            step(t, 0, True)
            step(t + 1, 1, True)
            return carry

        lax.fori_loop(0, _T // 2 - 1, body, 0)
        step(_T - 2, 0, False)
        step(_T - 1, 1, False)

        plsc.subcore_barrier()
        pltpu.sync_copy(acc.at[own], rb)
        pltpu.sync_copy(rb, out_hbm.at[c, own])

    return k(u, srcr, dstr)


def _sc_deg(srcr):
    f = 16
    mesh = plsc.VectorSubcoreMesh(core_axis_name="c", subcore_axis_name="s")

    @functools.partial(
        pl.kernel,
        out_type=jax.ShapeDtypeStruct((_NC, NP, f), jnp.float32),
        mesh=mesh,
        scratch_types=[
            pltpu.VMEM((_T, _CHUNK), jnp.int32),
            pltpu.VMEM((_CHUNK, f), jnp.float32),
            pltpu.VMEM((_RPT, f), jnp.float32),
            pltpu.VMEM_SHARED((NP, f), jnp.float32),
        ],
        compiler_params=pltpu.CompilerParams(use_tc_tiling_on_sc=False),
    )
    def k(src_hbm, out_hbm, idx_s, ones, rb, acc):
        c = lax.axis_index("c")
        s = lax.axis_index("s")
        wid = c * _NS + s
        own = pl.ds(s * _RPT, _RPT)

        _fill_const(ones, 1.0, _CHUNK, f)
        _fill_const(rb, 0.0, _RPT, f)
        pltpu.sync_copy(rb, acc.at[own])
        pltpu.sync_copy(src_hbm.at[wid], idx_s)
        plsc.subcore_barrier()

        def body(i, carry):
            pltpu.sync_copy(ones, acc.at[idx_s.at[i]], add=True)
            return carry

        lax.fori_loop(0, _T, body, 0)

        plsc.subcore_barrier()
        pltpu.sync_copy(acc.at[own], rb)
        pltpu.sync_copy(rb, out_hbm.at[c, own])

    return k(srcr)


def _mm1_body(x_ref, w_ref, b_ref, g_ref):
    g_ref[...] = lax.dot_general(
        x_ref[...], w_ref[...], (((1,), (0,)), ((), ())),
        preferred_element_type=jnp.float32) + b_ref[...]


def _prep_body(degp_ref, g_ref, dis_ref, loop_ref, u_ref):
    deg = degp_ref[0, :, 0:1] + degp_ref[1, :, 0:1]
    pos = deg > 0.0
    dis = jnp.where(pos, lax.rsqrt(jnp.maximum(deg, 1e-12)), 0.0)
    dis_ref[...] = dis
    loop_ref[...] = jnp.where(pos, 0.0, -1.0)
    u_ref[...] = dis * g_ref[...][:, 0:16]


def _post_mid_body(p_ref, g_ref, dis_ref, loop_ref, t_ref, u_ref):
    dis = dis_ref[...]
    g2 = g_ref[...][:, 0:16]
    lv = -dis * (p_ref[0] + p_ref[1]) + loop_ref[...] * g2
    t = g_ref[...][:, 16:32] + 2.0 * lv
    t_ref[...] = t
    u_ref[...] = dis * t


def _post_end_body(p_ref, g_ref, dis_ref, loop_ref, t_ref, h_ref, u_ref):
    dis = dis_ref[...]
    lv = -dis * (p_ref[0] + p_ref[1]) + loop_ref[...] * t_ref[...]
    h = jnp.maximum(g_ref[...][:, 32:48] + lv, 0.0)
    h_ref[...] = h
    u_ref[...] = dis * h


def _softmax_body(p_ref, g_ref, dis_ref, loop_ref, t_ref, out_ref):
    lv = -dis_ref[...] * (p_ref[0] + p_ref[1]) + loop_ref[...] * t_ref[...]
    logits = g_ref[...][:, 32:48] + lv
    valid = lax.broadcasted_iota(jnp.int32, logits.shape, 1) < NCLS
    logits = jnp.where(valid, logits, -1e30)
    m = jnp.max(logits, axis=1, keepdims=True)
    e = jnp.exp(logits - m)
    out_ref[...] = e / jnp.sum(e, axis=1, keepdims=True)


def _tx1_body(p_ref, h_ref, dis_ref, loop_ref, tx1_ref, u_ref):
    dis = dis_ref[...]
    tx1 = -dis * (p_ref[0] + p_ref[1]) + loop_ref[...] * h_ref[...]
    tx1_ref[...] = tx1
    u_ref[...] = dis * tx1


def _layer_mm_body(p_ref, h_ref, tx1_ref, dis_ref, loop_ref, w_ref, b_ref,
                   h_out_ref, u_ref):
    dis = dis_ref[...]
    h = h_ref[...]
    tx1 = tx1_ref[...]
    tx2 = 2.0 * (-dis * (p_ref[0] + p_ref[1]) + loop_ref[...] * tx1) - h
    hcat = jnp.concatenate([h, tx1, tx2], axis=1)
    out = lax.dot_general(hcat, w_ref[...], (((1,), (0,)), ((), ())),
                          preferred_element_type=jnp.float32) + b_ref[...]
    out = jnp.maximum(out, 0.0)
    h_out_ref[...] = out
    u_ref[...] = dis * out


def _layer_mm45_body(p_ref, h_ref, tx1_ref, dis_ref, loop_ref, w4_ref, b4_ref,
                     wc5_ref, bc5_ref, g5_ref, u_ref):
    dis = dis_ref[...]
    h = h_ref[...]
    tx1 = tx1_ref[...]
    tx2 = 2.0 * (-dis * (p_ref[0] + p_ref[1]) + loop_ref[...] * tx1) - h
    hcat = jnp.concatenate([h, tx1, tx2], axis=1)
    h4 = lax.dot_general(hcat, w4_ref[...], (((1,), (0,)), ((), ())),
                         preferred_element_type=jnp.float32) + b4_ref[...]
    h4 = jnp.maximum(h4, 0.0)
    g5 = lax.dot_general(h4, wc5_ref[...], (((1,), (0,)), ((), ())),
                         preferred_element_type=jnp.float32) + bc5_ref[...]
    g5_ref[...] = g5
    u_ref[...] = dis * g5[:, 0:16]


def _tc(body, out_shapes, *args, name):
    return pl.pallas_call(body, out_shape=out_shapes, name=name)(*args)


def _sds(shape):
    return jax.ShapeDtypeStruct(shape, jnp.float32)


def kernel(x, edge_index, W1, b1, W2, b2, W3, b3, W4, b4, W5, b5):
    srcr = edge_index[0].reshape(_NW, _T, _CHUNK)
    dstr = edge_index[1].reshape(_NW, _T, _CHUNK)
    xp = jnp.pad(x, ((0, NP - N), (0, 0)))

    wc1 = jnp.concatenate([W1[2], W1[1], W1[0] - W1[2]], axis=1)
    bc1 = jnp.concatenate(
        [jnp.zeros((32,), jnp.float32), b1]).reshape(1, 48)
    w5p = jnp.pad(W5, ((0, 0), (0, 0), (0, 16 - NCLS)))
    b5p = jnp.pad(b5, (0, 16 - NCLS))
    wc5 = jnp.concatenate([w5p[2], w5p[1], w5p[0] - w5p[2]], axis=1)
    bc5 = jnp.concatenate(
        [jnp.zeros((32,), jnp.float32), b5p]).reshape(1, 48)

    degp = _sc_deg(srcr)
    g = _tc(_mm1_body, _sds((NP, 48)), xp, wc1, bc1, name="tc_mm1")
    dis, loopv, u = _tc(_prep_body,
                        [_sds((NP, 1)), _sds((NP, 1)), _sds((NP, 16))],
                        degp, g, name="tc_prep")

    p = _sc_pass(u, srcr, dstr)
    t, u = _tc(_post_mid_body, [_sds((NP, 16))] * 2, p, g, dis, loopv,
               name="tc_post_mid1")
    p = _sc_pass(u, srcr, dstr)
    h, u = _tc(_post_end_body, [_sds((NP, 16))] * 2, p, g, dis, loopv, t,
               name="tc_post_end1")

    for li, (w, b) in enumerate(((W2, b2), (W3, b3), (W4, b4))):
        fin, fout = w.shape[1], w.shape[2]
        p = _sc_pass(u, srcr, dstr)
        tx1, u = _tc(_tx1_body, [_sds((NP, fin))] * 2, p, h, dis, loopv,
                     name=f"tc_tx1_l{li + 2}")
        p = _sc_pass(u, srcr, dstr)
        wstack = jnp.concatenate([w[0], w[1], w[2]], axis=0)
        if li < 2:
            h, u = _tc(_layer_mm_body, [_sds((NP, fout))] * 2,
                       p, h, tx1, dis, loopv, wstack, b.reshape(1, fout),
                       name=f"tc_mm_l{li + 2}")
        else:
            g, u = _tc(_layer_mm45_body, [_sds((NP, 48)), _sds((NP, 16))],
                       p, h, tx1, dis, loopv, wstack, b.reshape(1, fout),
                       wc5, bc5, name="tc_mm_l45")

    p = _sc_pass(u, srcr, dstr)
    t, u = _tc(_post_mid_body, [_sds((NP, 16))] * 2, p, g, dis, loopv,
               name="tc_post_mid5")
    p = _sc_pass(u, srcr, dstr)
    probs = _tc(_softmax_body, _sds((NP, 16)), p, g, dis, loopv, t,
                name="tc_softmax")
    return probs[:N, :NCLS]


# speedup vs baseline: 23.0018x; 1.0037x over previous
import functools

import jax
import jax.numpy as jnp
from jax import lax
from jax.experimental import pallas as pl
from jax.experimental.pallas import tpu as pltpu
from jax.experimental.pallas import tpu_sc as plsc

N = 10000
NP = 10240
E = 320000
NCLS = 13

_NC = 2
_NS = 16
_NW = _NC * _NS
_EPW = E // _NW
_CHUNK = 125
_T = _EPW // _CHUNK
_RPT = NP // _NS


def _fill_const(ref, val, nrow, f):
    cvec = jnp.full((16,), val, jnp.float32)

    def body(i, carry):
        for j in range(f // 16):
            ref[i, pl.ds(j * 16, 16)] = cvec
        return carry

    lax.fori_loop(0, nrow, body, 0)


def _sc_pass(u, srcr, dstr):
    f = u.shape[1]
    mesh = plsc.VectorSubcoreMesh(core_axis_name="c", subcore_axis_name="s")

    @functools.partial(
        pl.kernel,
        out_type=jax.ShapeDtypeStruct((_NC, NP, f), jnp.float32),
        mesh=mesh,
        scratch_types=[
            pltpu.VMEM((_T, _CHUNK), jnp.int32),
            pltpu.VMEM((_T, _CHUNK), jnp.int32),
            pltpu.VMEM((4, _CHUNK, f), jnp.float32),
            pltpu.VMEM((_RPT, f), jnp.float32),
            pltpu.VMEM_SHARED((NP, f), jnp.float32),
            [pltpu.SemaphoreType.DMA] * 4,
        ],
        compiler_params=pltpu.CompilerParams(use_tc_tiling_on_sc=False),
    )
    def k(u_hbm, src_hbm, dst_hbm, out_hbm, idx_s, idx_d, rows, rb, acc,
          gsems):
        c = lax.axis_index("c")
        s = lax.axis_index("s")
        wid = c * _NS + s
        own = pl.ds(s * _RPT, _RPT)

        _fill_const(rb, 0.0, _RPT, f)
        pltpu.sync_copy(rb, acc.at[own])

        pltpu.sync_copy(src_hbm.at[wid], idx_s)
        pltpu.sync_copy(dst_hbm.at[wid], idx_d)

        plsc.subcore_barrier()

        def issue_gather(i, b):
            pltpu.async_copy(u_hbm.at[idx_s.at[i]], rows.at[b], gsems[b])

        def wait_gather(i, b):
            pltpu.make_async_copy(u_hbm.at[idx_s.at[i]], rows.at[b],
                                  gsems[b]).wait()

        def scatter(i, b):
            pltpu.sync_copy(rows.at[b], acc.at[idx_d.at[i]], add=True)

        issue_gather(0, 0)
        issue_gather(1, 1)
        issue_gather(2, 2)

        def steady(i, b):
            wait_gather(i, b)
            scatter(i, b)
            issue_gather(i + 3, (b + 3) % 4)

        def tail(i, b):
            wait_gather(i, b)
            scatter(i, b)

        def body(kk, carry):
            i = 4 * kk
            steady(i, 0)
            steady(i + 1, 1)
            steady(i + 2, 2)
            steady(i + 3, 3)
            return carry

        lax.fori_loop(0, (_T - 4) // 4, body, 0)
        steady(_T - 4, 0)
        tail(_T - 3, 1)
        tail(_T - 2, 2)
        tail(_T - 1, 3)

        plsc.subcore_barrier()
        pltpu.sync_copy(acc.at[own], rb)
        pltpu.sync_copy(rb, out_hbm.at[c, own])

    return k(u, srcr, dstr)


def _sc_pass_wide(u, srcr, dstr):
    f = u.shape[1]
    if f <= 32:
        return _sc_pass(u, srcr, dstr)
    ps = [_sc_pass(u[:, j:j + 32], srcr, dstr) for j in range(0, f, 32)]
    return jnp.concatenate(ps, axis=2)


def _mm1_body(x_ref, w_ref, b_ref, g_ref):
    g_ref[...] = lax.dot_general(
        x_ref[...], w_ref[...], (((1,), (0,)), ((), ())),
        preferred_element_type=jnp.float32) + b_ref[...]


def _prep_body(degp_ref, g_ref, dis_ref, loop_ref, u_ref):
    deg = degp_ref[0, :, 0:1] + degp_ref[1, :, 0:1]
    pos = deg > 0.0
    dis = jnp.where(pos, lax.rsqrt(jnp.maximum(deg, 1e-12)), 0.0)
    dis_ref[...] = dis
    loop_ref[...] = jnp.where(pos, 0.0, -1.0)
    u_ref[...] = dis * g_ref[...][:, 0:16]


def _post_mid_body(p_ref, g_ref, dis_ref, loop_ref, t_ref, u_ref):
    dis = dis_ref[...]
    g2 = g_ref[...][:, 0:16]
    lv = -dis * (p_ref[0] + p_ref[1]) + loop_ref[...] * g2
    t = g_ref[...][:, 16:32] + 2.0 * lv
    t_ref[...] = t
    u_ref[...] = dis * t


def _post_end_body(p_ref, g_ref, dis_ref, loop_ref, t_ref, h_ref, u_ref):
    dis = dis_ref[...]
    lv = -dis * (p_ref[0] + p_ref[1]) + loop_ref[...] * t_ref[...]
    h = jnp.maximum(g_ref[...][:, 32:48] + lv, 0.0)
    h_ref[...] = h
    u_ref[...] = dis * h


def _softmax_body(p_ref, g_ref, dis_ref, loop_ref, t_ref, out_ref):
    lv = -dis_ref[...] * (p_ref[0] + p_ref[1]) + loop_ref[...] * t_ref[...]
    logits = g_ref[...][:, 32:48] + lv
    valid = lax.broadcasted_iota(jnp.int32, logits.shape, 1) < NCLS
    logits = jnp.where(valid, logits, -1e30)
    m = jnp.max(logits, axis=1, keepdims=True)
    e = jnp.exp(logits - m)
    out_ref[...] = e / jnp.sum(e, axis=1, keepdims=True)


def _tx1_body(p_ref, h_ref, dis_ref, loop_ref, tx1_ref, u_ref):
    dis = dis_ref[...]
    tx1 = -dis * (p_ref[0] + p_ref[1]) + loop_ref[...] * h_ref[...]
    tx1_ref[...] = tx1
    u_ref[...] = dis * tx1


def _layer_mm_body(p_ref, h_ref, tx1_ref, dis_ref, loop_ref, w_ref, b_ref,
                   h_out_ref, u_ref):
    dis = dis_ref[...]
    h = h_ref[...]
    tx1 = tx1_ref[...]
    tx2 = 2.0 * (-dis * (p_ref[0] + p_ref[1]) + loop_ref[...] * tx1) - h
    hcat = jnp.concatenate([h, tx1, tx2], axis=1)
    out = lax.dot_general(hcat, w_ref[...], (((1,), (0,)), ((), ())),
                          preferred_element_type=jnp.float32) + b_ref[...]
    out = jnp.maximum(out, 0.0)
    h_out_ref[...] = out
    u_ref[...] = dis * out


def _layer_mm45_body(p_ref, h_ref, tx1_ref, dis_ref, loop_ref, w4_ref, b4_ref,
                     wc5_ref, bc5_ref, g5_ref, u_ref):
    dis = dis_ref[...]
    h = h_ref[...]
    tx1 = tx1_ref[...]
    tx2 = 2.0 * (-dis * (p_ref[0] + p_ref[1]) + loop_ref[...] * tx1) - h
    hcat = jnp.concatenate([h, tx1, tx2], axis=1)
    h4 = lax.dot_general(hcat, w4_ref[...], (((1,), (0,)), ((), ())),
                         preferred_element_type=jnp.float32) + b4_ref[...]
    h4 = jnp.maximum(h4, 0.0)
    g5 = lax.dot_general(h4, wc5_ref[...], (((1,), (0,)), ((), ())),
                         preferred_element_type=jnp.float32) + bc5_ref[...]
    g5_ref[...] = g5
    u_ref[...] = dis * g5[:, 0:16]


def _tc(body, out_shapes, *args, name):
    return pl.pallas_call(body, out_shape=out_shapes, name=name)(*args)


def _sds(shape):
    return jax.ShapeDtypeStruct(shape, jnp.float32)


def kernel(x, edge_index, W1, b1, W2, b2, W3, b3, W4, b4, W5, b5):
    srcr = edge_index[0].reshape(_NW, _T, _CHUNK)
    dstr = edge_index[1].reshape(_NW, _T, _CHUNK)
    xp = jnp.pad(x, ((0, NP - N), (0, 0)))

    wc1 = jnp.concatenate([W1[2], W1[1], W1[0] - W1[2]], axis=1)
    bc1 = jnp.concatenate(
        [jnp.zeros((32,), jnp.float32), b1]).reshape(1, 48)
    w5p = jnp.pad(W5, ((0, 0), (0, 0), (0, 16 - NCLS)))
    b5p = jnp.pad(b5, (0, 16 - NCLS))
    wc5 = jnp.concatenate([w5p[2], w5p[1], w5p[0] - w5p[2]], axis=1)
    bc5 = jnp.concatenate(
        [jnp.zeros((32,), jnp.float32), b5p]).reshape(1, 48)

    ones = jnp.ones((NP, 16), jnp.float32)
    degp = _sc_pass(ones, srcr, srcr)
    g = _tc(_mm1_body, _sds((NP, 48)), xp, wc1, bc1, name="tc_mm1")
    dis, loopv, u = _tc(_prep_body,
                        [_sds((NP, 1)), _sds((NP, 1)), _sds((NP, 16))],
                        degp, g, name="tc_prep")

    p = _sc_pass_wide(u, srcr, dstr)
    t, u = _tc(_post_mid_body, [_sds((NP, 16))] * 2, p, g, dis, loopv,
               name="tc_post_mid1")
    p = _sc_pass_wide(u, srcr, dstr)
    h, u = _tc(_post_end_body, [_sds((NP, 16))] * 2, p, g, dis, loopv, t,
               name="tc_post_end1")

    for li, (w, b) in enumerate(((W2, b2), (W3, b3), (W4, b4))):
        fin, fout = w.shape[1], w.shape[2]
        p = _sc_pass_wide(u, srcr, dstr)
        tx1, u = _tc(_tx1_body, [_sds((NP, fin))] * 2, p, h, dis, loopv,
                     name=f"tc_tx1_l{li + 2}")
        p = _sc_pass_wide(u, srcr, dstr)
        wstack = jnp.concatenate([w[0], w[1], w[2]], axis=0)
        if li < 2:
            h, u = _tc(_layer_mm_body, [_sds((NP, fout))] * 2,
                       p, h, tx1, dis, loopv, wstack, b.reshape(1, fout),
                       name=f"tc_mm_l{li + 2}")
        else:
            g, u = _tc(_layer_mm45_body, [_sds((NP, 48)), _sds((NP, 16))],
                       p, h, tx1, dis, loopv, wstack, b.reshape(1, fout),
                       wc5, bc5, name="tc_mm_l45")

    p = _sc_pass_wide(u, srcr, dstr)
    t, u = _tc(_post_mid_body, [_sds((NP, 16))] * 2, p, g, dis, loopv,
               name="tc_post_mid5")
    p = _sc_pass_wide(u, srcr, dstr)
    probs = _tc(_softmax_body, _sds((NP, 16)), p, g, dis, loopv, t,
                name="tc_softmax")
    return probs[:N, :NCLS]


# speedup vs baseline: 24.7047x; 1.0740x over previous
import functools

import jax
import jax.numpy as jnp
from jax import lax
from jax.experimental import pallas as pl
from jax.experimental.pallas import tpu as pltpu
from jax.experimental.pallas import tpu_sc as plsc

N = 10000
NP = 10240
E = 320000
NCLS = 13

_NC = 2
_NS = 16
_NW = _NC * _NS
_EPW = E // _NW
_CHUNK = 125
_T = _EPW // _CHUNK
_RPT = NP // _NS


def _fill_const(ref, val, nrow, f):
    cvec = jnp.full((16,), val, jnp.float32)

    def body(i, carry):
        for j in range(f // 16):
            ref[i, pl.ds(j * 16, 16)] = cvec
        return carry

    lax.fori_loop(0, nrow, body, 0)


def _sc_pass(u, srcr, dstr):
    f = u.shape[1]
    mesh = plsc.VectorSubcoreMesh(core_axis_name="c", subcore_axis_name="s")

    @functools.partial(
        pl.kernel,
        out_type=jax.ShapeDtypeStruct((_NC, NP, f), jnp.float32),
        mesh=mesh,
        scratch_types=[
            pltpu.VMEM((_T, _CHUNK), jnp.int32),
            pltpu.VMEM((_T, _CHUNK), jnp.int32),
            pltpu.VMEM((4, _CHUNK, f), jnp.float32),
            pltpu.VMEM((_RPT, f), jnp.float32),
            pltpu.VMEM_SHARED((NP, f), jnp.float32),
            [pltpu.SemaphoreType.DMA] * 4,
            [pltpu.SemaphoreType.DMA] * 4,
        ],
        compiler_params=pltpu.CompilerParams(use_tc_tiling_on_sc=False),
    )
    def k(z_hbm, u_hbm, src_hbm, dst_hbm, out_hbm, idx_s, idx_d, rows, rb,
          acc, gsems, ssems):
        c = lax.axis_index("c")
        s = lax.axis_index("s")
        wid = c * _NS + s
        own = pl.ds(s * _RPT, _RPT)

        pltpu.sync_copy(z_hbm.at[own], acc.at[own])

        pltpu.sync_copy(src_hbm.at[wid], idx_s)
        pltpu.sync_copy(dst_hbm.at[wid], idx_d)

        plsc.subcore_barrier()

        def ig(i, b):
            pltpu.async_copy(u_hbm.at[idx_s.at[i]], rows.at[b], gsems[b])

        def wg(i, b):
            pltpu.make_async_copy(u_hbm.at[idx_s.at[i]], rows.at[b],
                                  gsems[b]).wait()

        def isc(i, b):
            pltpu.async_copy(rows.at[b], acc.at[idx_d.at[i]], ssems[b],
                             add=True)

        def wsc(i, b):
            pltpu.make_async_copy(rows.at[b], acc.at[idx_d.at[i]],
                                  ssems[b]).wait()

        ig(0, 0)
        ig(1, 1)
        ig(2, 2)
        wg(0, 0)
        isc(0, 0)
        ig(3, 3)
        wg(1, 1)
        isc(1, 1)

        def steady(i, b):
            b2 = (b + 2) % 4
            wsc(i - 2, b2)
            ig(i + 2, b2)
            wg(i, b)
            isc(i, b)

        def body(kk, carry):
            i = 4 * kk + 2
            steady(i, 2)
            steady(i + 1, 3)
            steady(i + 2, 0)
            steady(i + 3, 1)
            return carry

        lax.fori_loop(0, (_T - 4) // 4, body, 0)
        wg(_T - 2, 2)
        isc(_T - 2, 2)
        wg(_T - 1, 3)
        isc(_T - 1, 3)
        wsc(_T - 4, 0)
        wsc(_T - 3, 1)
        wsc(_T - 2, 2)
        wsc(_T - 1, 3)

        plsc.subcore_barrier()
        pltpu.sync_copy(acc.at[own], rb)
        pltpu.sync_copy(rb, out_hbm.at[c, own])

    return k(jnp.zeros((NP, f), jnp.float32), u, srcr, dstr)


def _sc_pass_wide(u, srcr, dstr):
    f = u.shape[1]
    if f <= 32:
        return _sc_pass(u, srcr, dstr)
    ps = [_sc_pass(u[:, j:j + 32], srcr, dstr) for j in range(0, f, 32)]
    return jnp.concatenate(ps, axis=2)


def _mm1_body(x_ref, w_ref, b_ref, g_ref):
    g_ref[...] = lax.dot_general(
        x_ref[...], w_ref[...], (((1,), (0,)), ((), ())),
        preferred_element_type=jnp.float32) + b_ref[...]


def _prep_body(degp_ref, g_ref, dis_ref, loop_ref, u_ref):
    deg = degp_ref[0, :, 0:1] + degp_ref[1, :, 0:1]
    pos = deg > 0.0
    dis = jnp.where(pos, lax.rsqrt(jnp.maximum(deg, 1e-12)), 0.0)
    dis_ref[...] = dis
    loop_ref[...] = jnp.where(pos, 0.0, -1.0)
    u_ref[...] = dis * g_ref[...][:, 0:16]


def _post_mid_body(p_ref, g_ref, dis_ref, loop_ref, t_ref, u_ref):
    dis = dis_ref[...]
    g2 = g_ref[...][:, 0:16]
    lv = -dis * (p_ref[0] + p_ref[1]) + loop_ref[...] * g2
    t = g_ref[...][:, 16:32] + 2.0 * lv
    t_ref[...] = t
    u_ref[...] = dis * t


def _post_end_body(p_ref, g_ref, dis_ref, loop_ref, t_ref, h_ref, u_ref):
    dis = dis_ref[...]
    lv = -dis * (p_ref[0] + p_ref[1]) + loop_ref[...] * t_ref[...]
    h = jnp.maximum(g_ref[...][:, 32:48] + lv, 0.0)
    h_ref[...] = h
    u_ref[...] = dis * h


def _softmax_body(p_ref, g_ref, dis_ref, loop_ref, t_ref, out_ref):
    lv = -dis_ref[...] * (p_ref[0] + p_ref[1]) + loop_ref[...] * t_ref[...]
    logits = g_ref[...][:, 32:48] + lv
    valid = lax.broadcasted_iota(jnp.int32, logits.shape, 1) < NCLS
    logits = jnp.where(valid, logits, -1e30)
    m = jnp.max(logits, axis=1, keepdims=True)
    e = jnp.exp(logits - m)
    out_ref[...] = e / jnp.sum(e, axis=1, keepdims=True)


def _tx1_body(p_ref, h_ref, dis_ref, loop_ref, tx1_ref, u_ref):
    dis = dis_ref[...]
    tx1 = -dis * (p_ref[0] + p_ref[1]) + loop_ref[...] * h_ref[...]
    tx1_ref[...] = tx1
    u_ref[...] = dis * tx1


def _layer_mm_body(p_ref, h_ref, tx1_ref, dis_ref, loop_ref, w_ref, b_ref,
                   h_out_ref, u_ref):
    dis = dis_ref[...]
    h = h_ref[...]
    tx1 = tx1_ref[...]
    tx2 = 2.0 * (-dis * (p_ref[0] + p_ref[1]) + loop_ref[...] * tx1) - h
    hcat = jnp.concatenate([h, tx1, tx2], axis=1)
    out = lax.dot_general(hcat, w_ref[...], (((1,), (0,)), ((), ())),
                          preferred_element_type=jnp.float32) + b_ref[...]
    out = jnp.maximum(out, 0.0)
    h_out_ref[...] = out
    u_ref[...] = dis * out


def _layer_mm45_body(p_ref, h_ref, tx1_ref, dis_ref, loop_ref, w4_ref, b4_ref,
                     wc5_ref, bc5_ref, g5_ref, u_ref):
    dis = dis_ref[...]
    h = h_ref[...]
    tx1 = tx1_ref[...]
    tx2 = 2.0 * (-dis * (p_ref[0] + p_ref[1]) + loop_ref[...] * tx1) - h
    hcat = jnp.concatenate([h, tx1, tx2], axis=1)
    h4 = lax.dot_general(hcat, w4_ref[...], (((1,), (0,)), ((), ())),
                         preferred_element_type=jnp.float32) + b4_ref[...]
    h4 = jnp.maximum(h4, 0.0)
    g5 = lax.dot_general(h4, wc5_ref[...], (((1,), (0,)), ((), ())),
                         preferred_element_type=jnp.float32) + bc5_ref[...]
    g5_ref[...] = g5
    u_ref[...] = dis * g5[:, 0:16]


def _tc(body, out_shapes, *args, name):
    return pl.pallas_call(body, out_shape=out_shapes, name=name)(*args)


def _sds(shape):
    return jax.ShapeDtypeStruct(shape, jnp.float32)


def kernel(x, edge_index, W1, b1, W2, b2, W3, b3, W4, b4, W5, b5):
    srcr = edge_index[0].reshape(_NW, _T, _CHUNK)
    dstr = edge_index[1].reshape(_NW, _T, _CHUNK)
    xp = jnp.pad(x, ((0, NP - N), (0, 0)))

    wc1 = jnp.concatenate([W1[2], W1[1], W1[0] - W1[2]], axis=1)
    bc1 = jnp.concatenate(
        [jnp.zeros((32,), jnp.float32), b1]).reshape(1, 48)
    w5p = jnp.pad(W5, ((0, 0), (0, 0), (0, 16 - NCLS)))
    b5p = jnp.pad(b5, (0, 16 - NCLS))
    wc5 = jnp.concatenate([w5p[2], w5p[1], w5p[0] - w5p[2]], axis=1)
    bc5 = jnp.concatenate(
        [jnp.zeros((32,), jnp.float32), b5p]).reshape(1, 48)

    ones = jnp.ones((NP, 16), jnp.float32)
    degp = _sc_pass(ones, srcr, srcr)
    g = _tc(_mm1_body, _sds((NP, 48)), xp, wc1, bc1, name="tc_mm1")
    dis, loopv, u = _tc(_prep_body,
                        [_sds((NP, 1)), _sds((NP, 1)), _sds((NP, 16))],
                        degp, g, name="tc_prep")

    p = _sc_pass_wide(u, srcr, dstr)
    t, u = _tc(_post_mid_body, [_sds((NP, 16))] * 2, p, g, dis, loopv,
               name="tc_post_mid1")
    p = _sc_pass_wide(u, srcr, dstr)
    h, u = _tc(_post_end_body, [_sds((NP, 16))] * 2, p, g, dis, loopv, t,
               name="tc_post_end1")

    for li, (w, b) in enumerate(((W2, b2), (W3, b3), (W4, b4))):
        fin, fout = w.shape[1], w.shape[2]
        p = _sc_pass_wide(u, srcr, dstr)
        tx1, u = _tc(_tx1_body, [_sds((NP, fin))] * 2, p, h, dis, loopv,
                     name=f"tc_tx1_l{li + 2}")
        p = _sc_pass_wide(u, srcr, dstr)
        wstack = jnp.concatenate([w[0], w[1], w[2]], axis=0)
        if li < 2:
            h, u = _tc(_layer_mm_body, [_sds((NP, fout))] * 2,
                       p, h, tx1, dis, loopv, wstack, b.reshape(1, fout),
                       name=f"tc_mm_l{li + 2}")
        else:
            g, u = _tc(_layer_mm45_body, [_sds((NP, 48)), _sds((NP, 16))],
                       p, h, tx1, dis, loopv, wstack, b.reshape(1, fout),
                       wc5, bc5, name="tc_mm_l45")

    p = _sc_pass_wide(u, srcr, dstr)
    t, u = _tc(_post_mid_body, [_sds((NP, 16))] * 2, p, g, dis, loopv,
               name="tc_post_mid5")
    p = _sc_pass_wide(u, srcr, dstr)
    probs = _tc(_softmax_body, _sds((NP, 16)), p, g, dis, loopv, t,
                name="tc_softmax")
    return probs[:N, :NCLS]


# speedup vs baseline: 24.7205x; 1.0006x over previous
import functools

import jax
import jax.numpy as jnp
from jax import lax
from jax.experimental import pallas as pl
from jax.experimental.pallas import tpu as pltpu
from jax.experimental.pallas import tpu_sc as plsc

N = 10000
NP = 10240
E = 320000
NCLS = 13

_NC = 2
_NS = 16
_NW = _NC * _NS
_EPW = E // _NW
_CHUNK = 125
_T = _EPW // _CHUNK
_RPT = NP // _NS


def _fill_const(ref, val, nrow, f):
    cvec = jnp.full((16,), val, jnp.float32)

    def body(i, carry):
        for j in range(f // 16):
            ref[i, pl.ds(j * 16, 16)] = cvec
        return carry

    lax.fori_loop(0, nrow, body, 0)


def _sc_pass(u, srcr, dstr):
    f = u.shape[1]
    mesh = plsc.VectorSubcoreMesh(core_axis_name="c", subcore_axis_name="s")

    @functools.partial(
        pl.kernel,
        out_type=jax.ShapeDtypeStruct((_NC, NP, f), jnp.float32),
        mesh=mesh,
        scratch_types=[
            pltpu.VMEM((_T, _CHUNK), jnp.int32),
            pltpu.VMEM((_T, _CHUNK), jnp.int32),
            pltpu.VMEM((4, _CHUNK, f), jnp.float32),
            pltpu.VMEM((_RPT, f), jnp.float32),
            pltpu.VMEM_SHARED((NP, f), jnp.float32),
            [pltpu.SemaphoreType.DMA] * 4,
            [pltpu.SemaphoreType.DMA] * 4,
        ],
        compiler_params=pltpu.CompilerParams(use_tc_tiling_on_sc=False),
    )
    def k(z_hbm, u_hbm, src_hbm, dst_hbm, out_hbm, idx_s, idx_d, rows, rb,
          acc, gsems, ssems):
        c = lax.axis_index("c")
        s = lax.axis_index("s")
        wid = c * _NS + s
        own = pl.ds(s * _RPT, _RPT)

        pltpu.sync_copy(z_hbm.at[own], acc.at[own])

        pltpu.sync_copy(src_hbm.at[wid], idx_s)
        pltpu.sync_copy(dst_hbm.at[wid], idx_d)

        plsc.subcore_barrier()

        def ig(i, b):
            pltpu.async_copy(u_hbm.at[idx_s.at[i]], rows.at[b], gsems[b])

        def wg(i, b):
            pltpu.make_async_copy(u_hbm.at[idx_s.at[i]], rows.at[b],
                                  gsems[b]).wait()

        def isc(i, b):
            pltpu.async_copy(rows.at[b], acc.at[idx_d.at[i]], ssems[b],
                             add=True)

        def wsc(i, b):
            pltpu.make_async_copy(rows.at[b], acc.at[idx_d.at[i]],
                                  ssems[b]).wait()

        ig(0, 0)
        ig(1, 1)
        ig(2, 2)
        wg(0, 0)
        isc(0, 0)
        ig(3, 3)
        wg(1, 1)
        isc(1, 1)

        def steady(i, b):
            b2 = (b + 2) % 4
            wsc(i - 2, b2)
            ig(i + 2, b2)
            wg(i, b)
            isc(i, b)

        def body(kk, carry):
            i = 4 * kk + 2
            steady(i, 2)
            steady(i + 1, 3)
            steady(i + 2, 0)
            steady(i + 3, 1)
            return carry

        lax.fori_loop(0, (_T - 4) // 4, body, 0)
        wg(_T - 2, 2)
        isc(_T - 2, 2)
        wg(_T - 1, 3)
        isc(_T - 1, 3)
        wsc(_T - 4, 0)
        wsc(_T - 3, 1)
        wsc(_T - 2, 2)
        wsc(_T - 1, 3)

        plsc.subcore_barrier()
        pltpu.sync_copy(acc.at[own], rb)
        pltpu.sync_copy(rb, out_hbm.at[c, own])

    return k(jnp.zeros((NP, f), jnp.float32), u, srcr, dstr)


def _sc_pass_wide(u, srcr, dstr):
    f = u.shape[1]
    if f <= 32:
        return _sc_pass(u, srcr, dstr)
    ps = [_sc_pass(u[:, j:j + 32], srcr, dstr) for j in range(0, f, 32)]
    return jnp.concatenate(ps, axis=2)


def _mm1_prep_body(x_ref, w_ref, b_ref, degp_ref, g_ref, dis_ref, loop_ref,
                   u_ref):
    g = lax.dot_general(
        x_ref[...], w_ref[...], (((1,), (0,)), ((), ())),
        preferred_element_type=jnp.float32) + b_ref[...]
    g_ref[...] = g
    deg = degp_ref[0, :, 0:1] + degp_ref[1, :, 0:1]
    pos = deg > 0.0
    dis = jnp.where(pos, lax.rsqrt(jnp.maximum(deg, 1e-12)), 0.0)
    dis_ref[...] = dis
    loop_ref[...] = jnp.where(pos, 0.0, -1.0)
    u_ref[...] = dis * g[:, 0:16]


def _post_mid_body(p_ref, g_ref, dis_ref, loop_ref, t_ref, u_ref):
    dis = dis_ref[...]
    g2 = g_ref[...][:, 0:16]
    lv = -dis * (p_ref[0] + p_ref[1]) + loop_ref[...] * g2
    t = g_ref[...][:, 16:32] + 2.0 * lv
    t_ref[...] = t
    u_ref[...] = dis * t


def _post_end_body(p_ref, g_ref, dis_ref, loop_ref, t_ref, h_ref, u_ref):
    dis = dis_ref[...]
    lv = -dis * (p_ref[0] + p_ref[1]) + loop_ref[...] * t_ref[...]
    h = jnp.maximum(g_ref[...][:, 32:48] + lv, 0.0)
    h_ref[...] = h
    u_ref[...] = dis * h


def _softmax_body(p_ref, g_ref, dis_ref, loop_ref, t_ref, out_ref):
    lv = -dis_ref[...] * (p_ref[0] + p_ref[1]) + loop_ref[...] * t_ref[...]
    logits = g_ref[...][:, 32:48] + lv
    valid = lax.broadcasted_iota(jnp.int32, logits.shape, 1) < NCLS
    logits = jnp.where(valid, logits, -1e30)
    m = jnp.max(logits, axis=1, keepdims=True)
    e = jnp.exp(logits - m)
    out_ref[...] = e / jnp.sum(e, axis=1, keepdims=True)


def _tx1_body(p_ref, h_ref, dis_ref, loop_ref, tx1_ref, u_ref):
    dis = dis_ref[...]
    tx1 = -dis * (p_ref[0] + p_ref[1]) + loop_ref[...] * h_ref[...]
    tx1_ref[...] = tx1
    u_ref[...] = dis * tx1


def _layer_mm_body(p_ref, h_ref, tx1_ref, dis_ref, loop_ref, w_ref, b_ref,
                   h_out_ref, u_ref):
    dis = dis_ref[...]
    h = h_ref[...]
    tx1 = tx1_ref[...]
    tx2 = 2.0 * (-dis * (p_ref[0] + p_ref[1]) + loop_ref[...] * tx1) - h
    hcat = jnp.concatenate([h, tx1, tx2], axis=1)
    out = lax.dot_general(hcat, w_ref[...], (((1,), (0,)), ((), ())),
                          preferred_element_type=jnp.float32) + b_ref[...]
    out = jnp.maximum(out, 0.0)
    h_out_ref[...] = out
    u_ref[...] = dis * out


def _layer_mm45_body(p_ref, h_ref, tx1_ref, dis_ref, loop_ref, w4_ref, b4_ref,
                     wc5_ref, bc5_ref, g5_ref, u_ref):
    dis = dis_ref[...]
    h = h_ref[...]
    tx1 = tx1_ref[...]
    tx2 = 2.0 * (-dis * (p_ref[0] + p_ref[1]) + loop_ref[...] * tx1) - h
    hcat = jnp.concatenate([h, tx1, tx2], axis=1)
    h4 = lax.dot_general(hcat, w4_ref[...], (((1,), (0,)), ((), ())),
                         preferred_element_type=jnp.float32) + b4_ref[...]
    h4 = jnp.maximum(h4, 0.0)
    g5 = lax.dot_general(h4, wc5_ref[...], (((1,), (0,)), ((), ())),
                         preferred_element_type=jnp.float32) + bc5_ref[...]
    g5_ref[...] = g5
    u_ref[...] = dis * g5[:, 0:16]


def _tc(body, out_shapes, *args, name):
    return pl.pallas_call(body, out_shape=out_shapes, name=name)(*args)


def _sds(shape):
    return jax.ShapeDtypeStruct(shape, jnp.float32)


def kernel(x, edge_index, W1, b1, W2, b2, W3, b3, W4, b4, W5, b5):
    srcr = edge_index[0].reshape(_NW, _T, _CHUNK)
    dstr = edge_index[1].reshape(_NW, _T, _CHUNK)
    xp = jnp.pad(x, ((0, NP - N), (0, 0)))

    wc1 = jnp.concatenate([W1[2], W1[1], W1[0] - W1[2]], axis=1)
    bc1 = jnp.concatenate(
        [jnp.zeros((32,), jnp.float32), b1]).reshape(1, 48)
    w5p = jnp.pad(W5, ((0, 0), (0, 0), (0, 16 - NCLS)))
    b5p = jnp.pad(b5, (0, 16 - NCLS))
    wc5 = jnp.concatenate([w5p[2], w5p[1], w5p[0] - w5p[2]], axis=1)
    bc5 = jnp.concatenate(
        [jnp.zeros((32,), jnp.float32), b5p]).reshape(1, 48)

    ones = jnp.ones((NP, 16), jnp.float32)
    degp = _sc_pass(ones, srcr, srcr)
    g, dis, loopv, u = _tc(
        _mm1_prep_body,
        [_sds((NP, 48)), _sds((NP, 1)), _sds((NP, 1)), _sds((NP, 16))],
        xp, wc1, bc1, degp, name="tc_mm1_prep")

    p = _sc_pass_wide(u, srcr, dstr)
    t, u = _tc(_post_mid_body, [_sds((NP, 16))] * 2, p, g, dis, loopv,
               name="tc_post_mid1")
    p = _sc_pass_wide(u, srcr, dstr)
    h, u = _tc(_post_end_body, [_sds((NP, 16))] * 2, p, g, dis, loopv, t,
               name="tc_post_end1")

    for li, (w, b) in enumerate(((W2, b2), (W3, b3), (W4, b4))):
        fin, fout = w.shape[1], w.shape[2]
        p = _sc_pass_wide(u, srcr, dstr)
        tx1, u = _tc(_tx1_body, [_sds((NP, fin))] * 2, p, h, dis, loopv,
                     name=f"tc_tx1_l{li + 2}")
        p = _sc_pass_wide(u, srcr, dstr)
        wstack = jnp.concatenate([w[0], w[1], w[2]], axis=0)
        if li < 2:
            h, u = _tc(_layer_mm_body, [_sds((NP, fout))] * 2,
                       p, h, tx1, dis, loopv, wstack, b.reshape(1, fout),
                       name=f"tc_mm_l{li + 2}")
        else:
            g, u = _tc(_layer_mm45_body, [_sds((NP, 48)), _sds((NP, 16))],
                       p, h, tx1, dis, loopv, wstack, b.reshape(1, fout),
                       wc5, bc5, name="tc_mm_l45")

    p = _sc_pass_wide(u, srcr, dstr)
    t, u = _tc(_post_mid_body, [_sds((NP, 16))] * 2, p, g, dis, loopv,
               name="tc_post_mid5")
    p = _sc_pass_wide(u, srcr, dstr)
    probs = _tc(_softmax_body, _sds((NP, 16)), p, g, dis, loopv, t,
                name="tc_softmax")
    return probs[:N, :NCLS]


# speedup vs baseline: 24.7205x; 1.0000x over previous
import functools

import jax
import jax.numpy as jnp
from jax import lax
from jax.experimental import pallas as pl
from jax.experimental.pallas import tpu as pltpu
from jax.experimental.pallas import tpu_sc as plsc

N = 10000
NP = 10240
E = 320000
NCLS = 13

_NC = 2
_NS = 16
_NW = _NC * _NS
_EPW = E // _NW
_CHUNK = 125
_T = _EPW // _CHUNK
_RPT = NP // _NS


def _sc_pass(u, srcr, dstr):
    f = u.shape[1]
    mesh = plsc.VectorSubcoreMesh(core_axis_name="c", subcore_axis_name="s")

    @functools.partial(
        pl.kernel,
        out_type=jax.ShapeDtypeStruct((_NC, NP, f), jnp.float32),
        mesh=mesh,
        scratch_types=[
            pltpu.VMEM((_T, _CHUNK), jnp.int32),
            pltpu.VMEM((_T, _CHUNK), jnp.int32),
            pltpu.VMEM((4, _CHUNK, f), jnp.float32),
            pltpu.VMEM((_RPT, f), jnp.float32),
            pltpu.VMEM_SHARED((NP, f), jnp.float32),
            [pltpu.SemaphoreType.DMA] * 4,
            [pltpu.SemaphoreType.DMA] * 4,
        ],
        compiler_params=pltpu.CompilerParams(use_tc_tiling_on_sc=False),
    )
    def k(z_hbm, u_hbm, src_hbm, dst_hbm, out_hbm, idx_s, idx_d, rows, rb,
          acc, gsems, ssems):
        c = lax.axis_index("c")
        s = lax.axis_index("s")
        wid = c * _NS + s
        own = pl.ds(s * _RPT, _RPT)

        pltpu.sync_copy(z_hbm.at[own], acc.at[own])

        pltpu.sync_copy(src_hbm.at[wid], idx_s)
        pltpu.sync_copy(dst_hbm.at[wid], idx_d)

        plsc.subcore_barrier()

        def ig(i, b):
            pltpu.async_copy(u_hbm.at[idx_s.at[i]], rows.at[b], gsems[b])

        def wg(i, b):
            pltpu.make_async_copy(u_hbm.at[idx_s.at[i]], rows.at[b],
                                  gsems[b]).wait()

        def isc(i, b):
            pltpu.async_copy(rows.at[b], acc.at[idx_d.at[i]], ssems[b],
                             add=True)

        def wsc(i, b):
            pltpu.make_async_copy(rows.at[b], acc.at[idx_d.at[i]],
                                  ssems[b]).wait()

        ig(0, 0)
        ig(1, 1)
        ig(2, 2)
        wg(0, 0)
        isc(0, 0)
        ig(3, 3)
        wg(1, 1)
        isc(1, 1)

        def steady(i, b):
            b2 = (b + 2) % 4
            wsc(i - 2, b2)
            ig(i + 2, b2)
            wg(i, b)
            isc(i, b)

        def body(kk, carry):
            i = 4 * kk + 2
            steady(i, 2)
            steady(i + 1, 3)
            steady(i + 2, 0)
            steady(i + 3, 1)
            return carry

        lax.fori_loop(0, (_T - 4) // 4, body, 0)
        wg(_T - 2, 2)
        isc(_T - 2, 2)
        wg(_T - 1, 3)
        isc(_T - 1, 3)
        wsc(_T - 4, 0)
        wsc(_T - 3, 1)
        wsc(_T - 2, 2)
        wsc(_T - 1, 3)

        plsc.subcore_barrier()
        pltpu.sync_copy(acc.at[own], rb)
        pltpu.sync_copy(rb, out_hbm.at[c, own])

    return k(jnp.zeros((NP, f), jnp.float32), u, srcr, dstr)


def _sc_pass_wide(u, srcr, dstr):
    f = u.shape[1]
    if f <= 32:
        return _sc_pass(u, srcr, dstr)
    ps = [_sc_pass(u[:, j:j + 32], srcr, dstr) for j in range(0, f, 32)]
    return jnp.concatenate(ps, axis=2)


def _mm1_prep_body(x_ref, w_ref, b_ref, degp_ref, g_ref, dis_ref, loop_ref,
                   u_ref):
    g = lax.dot_general(
        x_ref[...], w_ref[...], (((1,), (0,)), ((), ())),
        preferred_element_type=jnp.float32) + b_ref[...]
    g_ref[...] = g
    deg = degp_ref[0, :, 0:1] + degp_ref[1, :, 0:1]
    pos = deg > 0.0
    dis = jnp.where(pos, lax.rsqrt(jnp.maximum(deg, 1e-12)), 0.0)
    dis_ref[...] = dis
    loop_ref[...] = jnp.where(pos, 0.0, -1.0)
    u_ref[...] = dis * g[:, 0:16]


def _post_mid_body(p_ref, g_ref, dis_ref, loop_ref, t_ref, u_ref):
    dis = dis_ref[...]
    g2 = g_ref[...][:, 0:16]
    lv = -dis * (p_ref[0] + p_ref[1]) + loop_ref[...] * g2
    t = g_ref[...][:, 16:32] + 2.0 * lv
    t_ref[...] = t
    u_ref[...] = dis * t


def _post_end_body(p_ref, g_ref, dis_ref, loop_ref, t_ref, h_ref, u_ref):
    dis = dis_ref[...]
    lv = -dis * (p_ref[0] + p_ref[1]) + loop_ref[...] * t_ref[...]
    h = jnp.maximum(g_ref[...][:, 32:48] + lv, 0.0)
    h_ref[...] = h
    u_ref[...] = dis * h


def _softmax_body(p_ref, g_ref, dis_ref, loop_ref, t_ref, out_ref):
    lv = -dis_ref[...] * (p_ref[0] + p_ref[1]) + loop_ref[...] * t_ref[...]
    logits = g_ref[...][:, 32:48] + lv
    valid = lax.broadcasted_iota(jnp.int32, logits.shape, 1) < NCLS
    logits = jnp.where(valid, logits, -1e30)
    m = jnp.max(logits, axis=1, keepdims=True)
    e = jnp.exp(logits - m)
    out_ref[...] = e / jnp.sum(e, axis=1, keepdims=True)


def _tx1_body(p_ref, h_ref, dis_ref, loop_ref, tx1_ref, u_ref):
    dis = dis_ref[...]
    tx1 = -dis * (p_ref[0] + p_ref[1]) + loop_ref[...] * h_ref[...]
    tx1_ref[...] = tx1
    u_ref[...] = dis * tx1


def _layer_mm_body(p_ref, h_ref, tx1_ref, dis_ref, loop_ref, w_ref, b_ref,
                   h_out_ref, u_ref):
    dis = dis_ref[...]
    h = h_ref[...]
    tx1 = tx1_ref[...]
    tx2 = 2.0 * (-dis * (p_ref[0] + p_ref[1]) + loop_ref[...] * tx1) - h
    hcat = jnp.concatenate([h, tx1, tx2], axis=1)
    out = lax.dot_general(hcat, w_ref[...], (((1,), (0,)), ((), ())),
                          preferred_element_type=jnp.float32) + b_ref[...]
    out = jnp.maximum(out, 0.0)
    h_out_ref[...] = out
    u_ref[...] = dis * out


def _layer_mm45_body(p_ref, h_ref, tx1_ref, dis_ref, loop_ref, w4_ref, b4_ref,
                     wc5_ref, bc5_ref, g5_ref, u_ref):
    dis = dis_ref[...]
    h = h_ref[...]
    tx1 = tx1_ref[...]
    tx2 = 2.0 * (-dis * (p_ref[0] + p_ref[1]) + loop_ref[...] * tx1) - h
    hcat = jnp.concatenate([h, tx1, tx2], axis=1)
    h4 = lax.dot_general(hcat, w4_ref[...], (((1,), (0,)), ((), ())),
                         preferred_element_type=jnp.float32) + b4_ref[...]
    h4 = jnp.maximum(h4, 0.0)
    g5 = lax.dot_general(h4, wc5_ref[...], (((1,), (0,)), ((), ())),
                         preferred_element_type=jnp.float32) + bc5_ref[...]
    g5_ref[...] = g5
    u_ref[...] = dis * g5[:, 0:16]


def _tc(body, out_shapes, *args, name):
    return pl.pallas_call(body, out_shape=out_shapes, name=name)(*args)


def _sds(shape):
    return jax.ShapeDtypeStruct(shape, jnp.float32)


def kernel(x, edge_index, W1, b1, W2, b2, W3, b3, W4, b4, W5, b5):
    srcr = edge_index[0].reshape(_NW, _T, _CHUNK)
    dstr = edge_index[1].reshape(_NW, _T, _CHUNK)
    xp = jnp.pad(x, ((0, NP - N), (0, 0)))

    wc1 = jnp.concatenate([W1[2], W1[1], W1[0] - W1[2]], axis=1)
    bc1 = jnp.concatenate(
        [jnp.zeros((32,), jnp.float32), b1]).reshape(1, 48)
    w5p = jnp.pad(W5, ((0, 0), (0, 0), (0, 16 - NCLS)))
    b5p = jnp.pad(b5, (0, 16 - NCLS))
    wc5 = jnp.concatenate([w5p[2], w5p[1], w5p[0] - w5p[2]], axis=1)
    bc5 = jnp.concatenate(
        [jnp.zeros((32,), jnp.float32), b5p]).reshape(1, 48)

    ones = jnp.ones((NP, 16), jnp.float32)
    degp = _sc_pass(ones, srcr, srcr)
    g, dis, loopv, u = _tc(
        _mm1_prep_body,
        [_sds((NP, 48)), _sds((NP, 1)), _sds((NP, 1)), _sds((NP, 16))],
        xp, wc1, bc1, degp, name="tc_mm1_prep")

    p = _sc_pass_wide(u, srcr, dstr)
    t, u = _tc(_post_mid_body, [_sds((NP, 16))] * 2, p, g, dis, loopv,
               name="tc_post_mid1")
    p = _sc_pass_wide(u, srcr, dstr)
    h, u = _tc(_post_end_body, [_sds((NP, 16))] * 2, p, g, dis, loopv, t,
               name="tc_post_end1")

    for li, (w, b) in enumerate(((W2, b2), (W3, b3), (W4, b4))):
        fin, fout = w.shape[1], w.shape[2]
        p = _sc_pass_wide(u, srcr, dstr)
        tx1, u = _tc(_tx1_body, [_sds((NP, fin))] * 2, p, h, dis, loopv,
                     name=f"tc_tx1_l{li + 2}")
        p = _sc_pass_wide(u, srcr, dstr)
        wstack = jnp.concatenate([w[0], w[1], w[2]], axis=0)
        if li < 2:
            h, u = _tc(_layer_mm_body, [_sds((NP, fout))] * 2,
                       p, h, tx1, dis, loopv, wstack, b.reshape(1, fout),
                       name=f"tc_mm_l{li + 2}")
        else:
            g, u = _tc(_layer_mm45_body, [_sds((NP, 48)), _sds((NP, 16))],
                       p, h, tx1, dis, loopv, wstack, b.reshape(1, fout),
                       wc5, bc5, name="tc_mm_l45")

    p = _sc_pass_wide(u, srcr, dstr)
    t, u = _tc(_post_mid_body, [_sds((NP, 16))] * 2, p, g, dis, loopv,
               name="tc_post_mid5")
    p = _sc_pass_wide(u, srcr, dstr)
    probs = _tc(_softmax_body, _sds((NP, 16)), p, g, dis, loopv, t,
                name="tc_softmax")
    return probs[:N, :NCLS]
